# baseline (speedup 1.0000x reference)
import jax
import jax.numpy as jnp
import numpy as np
from jax.experimental import pallas as pl
from jax.experimental.pallas import tpu as pltpu


def _spiral_map(cen):
    return {
        0: [(slice(1, 3), (cen - 1, slice(cen, cen + 2))),
            (slice(3, 5), (slice(cen, cen + 2), cen + 1)),
            (slice(5, 7), (cen + 1, slice(cen - 1, cen + 1))),
            (slice(7, 9), (slice(cen - 1, cen + 1), cen - 1))],
        1: [(slice(9, 13), (cen - 2, slice(cen - 1, cen + 3))),
            (slice(13, 17), (slice(cen - 1, cen + 3), cen + 2)),
            (slice(17, 21), (cen + 2, slice(cen - 2, cen + 2))),
            (slice(21, 25), (slice(cen - 2, cen + 2), cen - 2))],
        2: [(slice(25, 31), (cen - 3, slice(cen - 2, cen + 4))),
            (slice(31, 37), (slice(cen - 2, cen + 4), cen + 3)),
            (slice(37, 43), (cen + 3, slice(cen - 3, cen + 3))),
            (slice(43, 49), (slice(cen - 3, cen + 3), cen - 3))],
        3: [(slice(49, 57), (cen - 4, slice(cen - 3, cen + 5))),
            (slice(57, 65), (slice(cen - 3, cen + 5), cen + 4)),
            (slice(65, 73), (cen + 4, slice(cen - 4, cen + 4))),
            (slice(73, 81), (slice(cen - 4, cen + 4), cen - 4))],
        4: [(slice(81, 91), (cen - 5, slice(cen - 4, cen + 6))),
            (slice(91, 101), (slice(cen - 4, cen + 6), cen + 5)),
            (slice(101, 111), (cen + 5, slice(cen - 5, cen + 5))),
            (slice(111, 121), (slice(cen - 5, cen + 5), cen - 5))],
    }


def _src_perm(h):
    cen = h // 2
    src = np.empty(h * h, np.int64)
    src[0] = cen * h + cen
    for i in range(cen):
        for dest, (ri, ci) in _spiral_map(cen).get(i, []):
            if isinstance(ri, slice):
                cells = [(r, ci) for r in range(ri.start, ri.stop)]
            else:
                cells = [(ri, c) for c in range(ci.start, ci.stop)]
            for k, (r, c) in enumerate(cells):
                src[dest.start + k] = r * h + c
    return src


_H = 11
_HW = _H * _H


_CEN = _H // 2

_SEGS = [("row", _CEN, _CEN, 0, 1)]
for _i in range(_CEN):
    for _dest, (_ri, _ci) in _spiral_map(_CEN).get(_i, []):
        _d0 = _dest.start
        if isinstance(_ri, slice):
            _SEGS.append(("col", _ri.start, _ci, _d0, _ri.stop - _ri.start))
        else:
            _SEGS.append(("row", _ri, _ci.start, _d0, _ci.stop - _ci.start))

_MAXL = max(s[4] for s in _SEGS)
_NR = 6
_LAG = 3
_NHALF = 4


def _seg_body(x_ref, o_ref, *scratch):
    bufs, insem, outsem = scratch[:-2], scratch[-2], scratch[-1]
    b = x_ref.shape[2]
    hb = b // _NHALF
    units = []
    for kind, a0, a1, d0, ln in _SEGS:
        for hlf in range(_NHALF):
            units.append((kind, a0, a1, d0, ln, hlf * hb))
    n = len(units)

    def src_slc(u):
        kind, a0, a1, d0, ln, h0 = u
        if kind == "row":
            return x_ref.at[a0, pl.ds(a1, ln), pl.ds(h0, hb)]
        return x_ref.at[pl.ds(a0, ln), a1, pl.ds(h0, hb)]

    def in_cp(t):
        u = units[t]
        return pltpu.make_async_copy(
            src_slc(u), bufs[t % _NR].at[pl.ds(0, u[4])], insem.at[t % _NR])

    def out_cp(t):
        u = units[t]
        return pltpu.make_async_copy(
            bufs[t % _NR].at[pl.ds(0, u[4])],
            o_ref.at[pl.ds(u[3], u[4]), pl.ds(u[5], hb)], outsem.at[t % _NR])

    for t in range(n + _LAG):
        if t < n:
            if t >= _NR:
                out_cp(t - _NR).wait()
            in_cp(t).start()
        v = t - _LAG
        if 0 <= v < n:
            in_cp(v).wait()
            out_cp(v).start()
    for v in range(max(0, n - _NR), n):
        out_cp(v).wait()


def kernel(x):
    b, c, h, w = x.shape
    hw = h * w
    x4 = jnp.transpose(x, (2, 3, 0, 1))
    hb = b // _NHALF
    out_p = pl.pallas_call(
        _seg_body,
        grid=(1,),
        in_specs=[pl.BlockSpec(memory_space=pltpu.MemorySpace.HBM)],
        out_specs=pl.BlockSpec(memory_space=pltpu.MemorySpace.HBM),
        out_shape=jax.ShapeDtypeStruct((hw, b, c), x.dtype),
        scratch_shapes=[pltpu.VMEM((_MAXL, hb, c), x.dtype) for _ in range(_NR)]
        + [pltpu.SemaphoreType.DMA((_NR,)), pltpu.SemaphoreType.DMA((_NR,))],
    )(x4)
    return jnp.transpose(out_p, (1, 0, 2))


# speedup vs baseline: 1.0347x; 1.0347x over previous
import jax
import jax.numpy as jnp
import numpy as np
from jax.experimental import pallas as pl
from jax.experimental.pallas import tpu as pltpu


def _spiral_map(cen):
    return {
        0: [(slice(1, 3), (cen - 1, slice(cen, cen + 2))),
            (slice(3, 5), (slice(cen, cen + 2), cen + 1)),
            (slice(5, 7), (cen + 1, slice(cen - 1, cen + 1))),
            (slice(7, 9), (slice(cen - 1, cen + 1), cen - 1))],
        1: [(slice(9, 13), (cen - 2, slice(cen - 1, cen + 3))),
            (slice(13, 17), (slice(cen - 1, cen + 3), cen + 2)),
            (slice(17, 21), (cen + 2, slice(cen - 2, cen + 2))),
            (slice(21, 25), (slice(cen - 2, cen + 2), cen - 2))],
        2: [(slice(25, 31), (cen - 3, slice(cen - 2, cen + 4))),
            (slice(31, 37), (slice(cen - 2, cen + 4), cen + 3)),
            (slice(37, 43), (cen + 3, slice(cen - 3, cen + 3))),
            (slice(43, 49), (slice(cen - 3, cen + 3), cen - 3))],
        3: [(slice(49, 57), (cen - 4, slice(cen - 3, cen + 5))),
            (slice(57, 65), (slice(cen - 3, cen + 5), cen + 4)),
            (slice(65, 73), (cen + 4, slice(cen - 4, cen + 4))),
            (slice(73, 81), (slice(cen - 4, cen + 4), cen - 4))],
        4: [(slice(81, 91), (cen - 5, slice(cen - 4, cen + 6))),
            (slice(91, 101), (slice(cen - 4, cen + 6), cen + 5)),
            (slice(101, 111), (cen + 5, slice(cen - 5, cen + 5))),
            (slice(111, 121), (slice(cen - 5, cen + 5), cen - 5))],
    }


def _src_perm(h):
    cen = h // 2
    src = np.empty(h * h, np.int64)
    src[0] = cen * h + cen
    for i in range(cen):
        for dest, (ri, ci) in _spiral_map(cen).get(i, []):
            if isinstance(ri, slice):
                cells = [(r, ci) for r in range(ri.start, ri.stop)]
            else:
                cells = [(ri, c) for c in range(ci.start, ci.stop)]
            for k, (r, c) in enumerate(cells):
                src[dest.start + k] = r * h + c
    return src


_H = 11
_HW = _H * _H


_CEN = _H // 2

_SEGS = [("row", _CEN, _CEN, 0, 1)]
for _i in range(_CEN):
    for _dest, (_ri, _ci) in _spiral_map(_CEN).get(_i, []):
        _d0 = _dest.start
        if isinstance(_ri, slice):
            _SEGS.append(("col", _ri.start, _ci, _d0, _ri.stop - _ri.start))
        else:
            _SEGS.append(("row", _ri, _ci.start, _d0, _ci.stop - _ci.start))

_MAXL = max(s[4] for s in _SEGS)
_NR = 4
_LAG = 2
_NHALF = 1


def _seg_body(x_ref, o_ref, *scratch):
    bufs, insem, outsem = scratch[:-2], scratch[-2], scratch[-1]
    b = x_ref.shape[2]
    hb = b // _NHALF
    units = []
    for kind, a0, a1, d0, ln in _SEGS:
        for hlf in range(_NHALF):
            units.append((kind, a0, a1, d0, ln, hlf * hb))
    n = len(units)

    def src_slc(u):
        kind, a0, a1, d0, ln, h0 = u
        if kind == "row":
            return x_ref.at[a0, pl.ds(a1, ln), pl.ds(h0, hb)]
        return x_ref.at[pl.ds(a0, ln), a1, pl.ds(h0, hb)]

    def in_cp(t):
        u = units[t]
        return pltpu.make_async_copy(
            src_slc(u), bufs[t % _NR].at[pl.ds(0, u[4])], insem.at[t % _NR])

    def out_cp(t):
        u = units[t]
        return pltpu.make_async_copy(
            bufs[t % _NR].at[pl.ds(0, u[4])],
            o_ref.at[pl.ds(u[3], u[4]), pl.ds(u[5], hb)], outsem.at[t % _NR])

    for t in range(n + _LAG):
        if t < n:
            if t >= _NR:
                out_cp(t - _NR).wait()
            in_cp(t).start()
        v = t - _LAG
        if 0 <= v < n:
            in_cp(v).wait()
            out_cp(v).start()
    for v in range(max(0, n - _NR), n):
        out_cp(v).wait()


def kernel(x):
    b, c, h, w = x.shape
    hw = h * w
    x4 = jnp.transpose(x, (2, 3, 0, 1))
    hb = b // _NHALF
    out_p = pl.pallas_call(
        _seg_body,
        grid=(1,),
        in_specs=[pl.BlockSpec(memory_space=pltpu.MemorySpace.HBM)],
        out_specs=pl.BlockSpec(memory_space=pltpu.MemorySpace.HBM),
        out_shape=jax.ShapeDtypeStruct((hw, b, c), x.dtype),
        scratch_shapes=[pltpu.VMEM((_MAXL, hb, c), x.dtype) for _ in range(_NR)]
        + [pltpu.SemaphoreType.DMA((_NR,)), pltpu.SemaphoreType.DMA((_NR,))],
    )(x4)
    return jnp.transpose(out_p, (1, 0, 2))


# speedup vs baseline: 1.0403x; 1.0054x over previous
import jax
import jax.numpy as jnp
import numpy as np
from jax.experimental import pallas as pl
from jax.experimental.pallas import tpu as pltpu


def _spiral_map(cen):
    return {
        0: [(slice(1, 3), (cen - 1, slice(cen, cen + 2))),
            (slice(3, 5), (slice(cen, cen + 2), cen + 1)),
            (slice(5, 7), (cen + 1, slice(cen - 1, cen + 1))),
            (slice(7, 9), (slice(cen - 1, cen + 1), cen - 1))],
        1: [(slice(9, 13), (cen - 2, slice(cen - 1, cen + 3))),
            (slice(13, 17), (slice(cen - 1, cen + 3), cen + 2)),
            (slice(17, 21), (cen + 2, slice(cen - 2, cen + 2))),
            (slice(21, 25), (slice(cen - 2, cen + 2), cen - 2))],
        2: [(slice(25, 31), (cen - 3, slice(cen - 2, cen + 4))),
            (slice(31, 37), (slice(cen - 2, cen + 4), cen + 3)),
            (slice(37, 43), (cen + 3, slice(cen - 3, cen + 3))),
            (slice(43, 49), (slice(cen - 3, cen + 3), cen - 3))],
        3: [(slice(49, 57), (cen - 4, slice(cen - 3, cen + 5))),
            (slice(57, 65), (slice(cen - 3, cen + 5), cen + 4)),
            (slice(65, 73), (cen + 4, slice(cen - 4, cen + 4))),
            (slice(73, 81), (slice(cen - 4, cen + 4), cen - 4))],
        4: [(slice(81, 91), (cen - 5, slice(cen - 4, cen + 6))),
            (slice(91, 101), (slice(cen - 4, cen + 6), cen + 5)),
            (slice(101, 111), (cen + 5, slice(cen - 5, cen + 5))),
            (slice(111, 121), (slice(cen - 5, cen + 5), cen - 5))],
    }


def _src_perm(h):
    cen = h // 2
    src = np.empty(h * h, np.int64)
    src[0] = cen * h + cen
    for i in range(cen):
        for dest, (ri, ci) in _spiral_map(cen).get(i, []):
            if isinstance(ri, slice):
                cells = [(r, ci) for r in range(ri.start, ri.stop)]
            else:
                cells = [(ri, c) for c in range(ci.start, ci.stop)]
            for k, (r, c) in enumerate(cells):
                src[dest.start + k] = r * h + c
    return src


_H = 11
_HW = _H * _H


_CEN = _H // 2

_SEGS = [("row", _CEN, _CEN, 0, 1)]
for _i in range(_CEN):
    for _dest, (_ri, _ci) in _spiral_map(_CEN).get(_i, []):
        _d0 = _dest.start
        if isinstance(_ri, slice):
            _SEGS.append(("col", _ri.start, _ci, _d0, _ri.stop - _ri.start))
        else:
            _SEGS.append(("row", _ri, _ci.start, _d0, _ci.stop - _ci.start))

_MAXL = max(s[4] for s in _SEGS)
_NR = 5
_LAG = 2
_NHALF = 1


def _seg_body(x_ref, o_ref, *scratch):
    bufs, insem, outsem = scratch[:-2], scratch[-2], scratch[-1]
    b = x_ref.shape[2]
    hb = b // _NHALF
    units = []
    for kind, a0, a1, d0, ln in _SEGS:
        for hlf in range(_NHALF):
            units.append((kind, a0, a1, d0, ln, hlf * hb))
    n = len(units)

    def src_slc(u):
        kind, a0, a1, d0, ln, h0 = u
        if kind == "row":
            return x_ref.at[a0, pl.ds(a1, ln), pl.ds(h0, hb)]
        return x_ref.at[pl.ds(a0, ln), a1, pl.ds(h0, hb)]

    def in_cp(t):
        u = units[t]
        return pltpu.make_async_copy(
            src_slc(u), bufs[t % _NR].at[pl.ds(0, u[4])], insem.at[t % _NR])

    def out_cp(t):
        u = units[t]
        return pltpu.make_async_copy(
            bufs[t % _NR].at[pl.ds(0, u[4])],
            o_ref.at[pl.ds(u[3], u[4]), pl.ds(u[5], hb)], outsem.at[t % _NR])

    for t in range(n + _LAG):
        if t < n:
            if t >= _NR:
                out_cp(t - _NR).wait()
            in_cp(t).start()
        v = t - _LAG
        if 0 <= v < n:
            in_cp(v).wait()
            out_cp(v).start()
    for v in range(max(0, n - _NR), n):
        out_cp(v).wait()


def kernel(x):
    b, c, h, w = x.shape
    hw = h * w
    x4 = jnp.transpose(x, (2, 3, 0, 1))
    hb = b // _NHALF
    out_p = pl.pallas_call(
        _seg_body,
        grid=(1,),
        in_specs=[pl.BlockSpec(memory_space=pltpu.MemorySpace.HBM)],
        out_specs=pl.BlockSpec(memory_space=pltpu.MemorySpace.HBM),
        out_shape=jax.ShapeDtypeStruct((hw, b, c), x.dtype),
        scratch_shapes=[pltpu.VMEM((_MAXL, hb, c), x.dtype) for _ in range(_NR)]
        + [pltpu.SemaphoreType.DMA((_NR,)), pltpu.SemaphoreType.DMA((_NR,))],
    )(x4)
    return jnp.transpose(out_p, (1, 0, 2))
